# baseline (speedup 1.0000x reference)
import functools

import jax
import jax.numpy as jnp
from jax import lax
from jax.experimental import pallas as pl
from jax.experimental.pallas import tpu as pltpu

NFS = (16, 36, 56)
SBODY = 12
CAPS_E = (384, 416, 320)
BASES_E = (0, 384, 800)
NPATCH = 1024
ROWS = 40
TOPPAD = 8


def _compose_convs(w1, w2):
    k2 = w2.shape[2]
    data = w1.transpose(1, 0, 2, 3)
    kern = jnp.flip(w2, (2, 3))
    out = lax.conv_general_dilated(
        data, kern, (1, 1), [(k2 - 1, k2 - 1)] * 2,
        dimension_numbers=('NCHW', 'OIHW', 'NCHW'))
    return out.transpose(1, 0, 2, 3)


def _conv_ref(x, w, b, stride=1, pad=0):
    y = lax.conv_general_dilated(x, w, (stride, stride), [(pad, pad)] * 2,
                                 dimension_numbers=('NCHW', 'OIHW', 'NCHW'))
    return y + b[None, :, None, None]


def _deconv_ref(x, w, b):
    k = w.shape[2]
    wt = jnp.flip(w, (2, 3)).transpose(1, 0, 2, 3)
    lo = k - 1 - 4
    hi = k - 1 - 4 + 3
    y = lax.conv_general_dilated(x, wt, (1, 1), [(lo, hi), (lo, hi)],
                                 lhs_dilation=(4, 4),
                                 dimension_numbers=('NCHW', 'OIHW', 'NCHW'))
    return y + b[None, :, None, None]


def _toeplitz(w):
    n_out, c_in, k, _ = w.shape
    p = (k - 1) // 2
    wp = jnp.arange(32)
    kx = wp[:, None] - wp[None, :] + p
    mask = (kx >= 0) & (kx < k)
    kxc = jnp.clip(kx, 0, k - 1)
    g = w[:, :, :, kxc]
    g = g * mask[None, None, None]
    g = g.transpose(2, 3, 1, 4, 0)
    return g.reshape(k, 32 * c_in, 32 * n_out)


def _toeplitz_deconv(wt2):
    wp = jnp.arange(32)
    e = wp[:, None] - wp[None, :]
    mats = {}
    for i in range(4):
        for d in (-1, 0, 1):
            kh = 4 + i - 4 * d
            if not (0 <= kh <= 8):
                continue
            cols = []
            for j in range(4):
                kw = 4 + j - 4 * e
                m = (kw >= 0) & (kw <= 8)
                kwc = jnp.clip(kw, 0, 8)
                g = wt2[:, :, kh, :][:, :, kwc]
                g = g * m[None, None]
                cols.append(g.transpose(2, 0, 1, 3))
            g4 = jnp.stack(cols, axis=-1)
            mats[(i, d)] = g4.reshape(384, 3, 128).reshape(384, 384)
    return mats


def _prep_expert(p, nf):
    out = {}
    out['th'] = _toeplitz(p['head_w'])
    out['hb'] = jnp.tile(p['head_b'], 32)[None, :]
    out['ha'] = jnp.tile(p['head_a'], 32)[None, :]
    w0 = p['b0_w'][:, :, 0, 0]
    out['tb0'] = jnp.kron(jnp.eye(32, dtype=w0.dtype), w0.T)
    out['b0b'] = jnp.tile(p['b0_b'], 32)[None, :]
    for li in (1, 2, 3):
        out['t%d' % li] = _toeplitz(p['b%d_w' % li])
        out['b%db' % li] = jnp.tile(p['b%d_b' % li], 32)[None, :]
    out['t4'] = _toeplitz(p['b4_w'])
    out['b4b'] = jnp.tile(p['b4_b'], 32)[None, :]
    out['b4a'] = jnp.tile(p['b4_a'], 32)[None, :]
    wt2 = jnp.einsum('fs,fnij->snij', p['b6_w'][:, :, 0, 0], p['tail_w'])
    tdec = _toeplitz_deconv(wt2)
    keys = sorted(tdec)
    out['tt'] = jnp.stack([tdec[k] for k in keys])
    out['tt_keys'] = keys
    cst = jnp.broadcast_to(p['b6_b'][None, :, None, None],
                           (1, nf, 32, 32)).astype(jnp.float32)
    mb = _deconv_ref(cst, p['tail_w'], p['tail_b'])[0]
    out['mb'] = mb.reshape(3, 32, 512)
    return out


def _cls_body(x_ref, w0, b0, w1, b1, w2, b2, w3, b3, w4, b4, fcw, fcb, o_ref):
    z = jnp.dot(x_ref[...], w0[...], preferred_element_type=jnp.float32, precision=lax.Precision.HIGHEST)
    z = z + b0[...]
    z = jnp.where(z > 0, z, 0.1 * z)
    for w, b in ((w1, b1), (w2, b2), (w3, b3)):
        z = jnp.dot(z, w[...], preferred_element_type=jnp.float32, precision=lax.Precision.HIGHEST) + b[...]
        z = jnp.where(z > 0, z, 0.1 * z)
    z = jnp.dot(z, w4[...], preferred_element_type=jnp.float32, precision=lax.Precision.HIGHEST) + b4[...]
    r_patch = lax.broadcasted_iota(jnp.int32, (32, 2048), 1) // 64
    p_row = lax.broadcasted_iota(jnp.int32, (32, 2048), 0)
    avg = jnp.where(r_patch == p_row, 1.0 / 64.0, 0.0)
    g = jnp.dot(avg, z, preferred_element_type=jnp.float32, precision=lax.Precision.HIGHEST)
    o_ref[...] = jnp.dot(g, fcw[...],
                         preferred_element_type=jnp.float32, precision=lax.Precision.HIGHEST) + fcb[...]


def _classifier(x, cp):
    xc = x.reshape(NPATCH, 3, 8, 4, 8, 4).transpose(0, 2, 4, 1, 3, 5)
    xc = xc.reshape(NPATCH * 64, 48)
    ws = [cp['c0_w'].reshape(128, 48).T, cp['c0_b'][None, :],
          cp['c1_w'][:, :, 0, 0].T, cp['c1_b'][None, :],
          cp['c2_w'][:, :, 0, 0].T, cp['c2_b'][None, :],
          cp['c3_w'][:, :, 0, 0].T, cp['c3_b'][None, :],
          cp['c4_w'][:, :, 0, 0].T, cp['c4_b'][None, :],
          cp['fc_w'].T, cp['fc_b'][None, :]]
    specs = [pl.BlockSpec((2048, 48), lambda i: (i, 0))]
    for w in ws:
        specs.append(pl.BlockSpec(w.shape, lambda i: (0, 0)))
    return pl.pallas_call(
        _cls_body,
        grid=(32,),
        in_specs=specs,
        out_specs=pl.BlockSpec((32, 3), lambda i: (i, 0)),
        out_shape=jax.ShapeDtypeStruct((NPATCH, 3), jnp.float32),
    )(xc, *ws)


def _route_body(lg_ref, idx_ref, flg_ref):
    lg = lg_ref[...]
    l0, l1, l2 = lg[:, 0:1], lg[:, 1:2], lg[:, 2:3]
    e = jnp.where(l1 > l0, 1, 0)
    e = jnp.where(l2 > jnp.maximum(l0, l1), 2, e)
    lane3 = lax.broadcasted_iota(jnp.int32, (NPATCH, 3), 1)
    onehot = jnp.where(lane3 == e, 1.0, 0.0)
    r_lo = lax.broadcasted_iota(jnp.int32, (NPATCH, NPATCH), 0)
    c_lo = lax.broadcasted_iota(jnp.int32, (NPATCH, NPATCH), 1)
    ltri = jnp.where(r_lo >= c_lo, 1.0, 0.0)
    cum = jnp.dot(ltri, onehot, preferred_element_type=jnp.float32, precision=lax.Precision.HIGHEST)
    rank = jnp.sum(cum * onehot, axis=1, keepdims=True) - 1.0
    cnt = cum[NPATCH - 1:NPATCH, :]
    l3 = lax.broadcasted_iota(jnp.int32, (1, 3), 1)
    caps = jnp.where(l3 == 0, float(CAPS_E[0]),
                     jnp.where(l3 == 1, float(CAPS_E[1]), float(CAPS_E[2])))
    bases = jnp.where(l3 == 0, float(BASES_E[0]),
                      jnp.where(l3 == 1, float(BASES_E[1]), float(BASES_E[2])))
    mincnt = jnp.minimum(cnt, caps)
    cap_p = jnp.sum(onehot * caps, axis=1, keepdims=True)
    base_p = jnp.sum(onehot * bases, axis=1, keepdims=True)
    valid = jnp.where(rank < cap_p, 1.0, 0.0)
    slot_ok = base_p + rank
    dropped = 1.0 - valid
    kdrop = jnp.dot(ltri, dropped, preferred_element_type=jnp.float32, precision=lax.Precision.HIGHEST) - dropped
    spare = caps - mincnt
    s0 = spare[0, 0]
    s01 = spare[0, 0] + spare[0, 1]
    slot_sp = jnp.where(
        kdrop < s0, bases[0, 0] + mincnt[0, 0] + kdrop,
        jnp.where(kdrop < s01, bases[0, 1] + mincnt[0, 1] + (kdrop - s0),
                  bases[0, 2] + mincnt[0, 2] + (kdrop - s01)))
    slot = jnp.where(valid > 0, slot_ok, slot_sp)
    tlane = lax.broadcasted_iota(jnp.int32, (NPATCH, 1120), 1)
    sc = jnp.where(tlane == slot.astype(jnp.int32), 1.0, 0.0)
    pcol = lax.broadcasted_iota(jnp.int32, (NPATCH, 1), 0).astype(jnp.float32)
    idx = jnp.dot(pcol.T, sc, preferred_element_type=jnp.float32, precision=lax.Precision.HIGHEST)
    covered = jnp.dot(jnp.ones((1, NPATCH), jnp.float32), sc,
                      preferred_element_type=jnp.float32, precision=lax.Precision.HIGHEST)
    flg = jnp.dot(valid.T, sc, preferred_element_type=jnp.float32, precision=lax.Precision.HIGHEST)
    idx = idx + (1.0 - covered) * float(NPATCH)
    idx_ref[...] = jnp.broadcast_to(idx, (8, 1120))
    flg_ref[...] = jnp.broadcast_to(flg, (8, 1120))


def _route(logits):
    idx, flg = pl.pallas_call(
        _route_body,
        in_specs=[pl.BlockSpec((NPATCH, 3), lambda: (0, 0))],
        out_specs=[pl.BlockSpec((8, 1120), lambda: (0, 0)),
                   pl.BlockSpec((8, 1120), lambda: (0, 0))],
        out_shape=[jax.ShapeDtypeStruct((8, 1120), jnp.float32),
                   jax.ShapeDtypeStruct((8, 1120), jnp.float32)],
    )(logits)
    return idx[0].astype(jnp.int32), flg[0]


def _expert_body(nf, tt_keys, idx_ref, x_ref, th, hb, ha, tb0, b0b,
                 t1, b1b, t2, b2b, t3, b3b, t4, b4b, b4a, tt, mb, flg,
                 oprev_ref, o_ref, x0, buf, buf2, buf3):
    del oprev_ref

    @pl.when(pl.program_id(0) == 0)
    def _init():
        x0[...] = jnp.zeros_like(x0)
        buf[...] = jnp.zeros_like(buf)
        buf2[...] = jnp.zeros_like(buf2)
        buf3[...] = jnp.zeros_like(buf3)

    x0[TOPPAD:TOPPAD + 32, :] = x_ref[0]

    rmask = lax.broadcasted_iota(jnp.int32, (ROWS, 1), 0) < 32
    rmaskf = jnp.where(rmask, 1.0, 0.0)

    z = jnp.zeros((ROWS, 32 * nf), jnp.float32)
    for d in range(5):
        xs = x0[TOPPAD + d - 2:TOPPAD + d - 2 + ROWS, :]
        z = z + jnp.dot(xs, th[d], preferred_element_type=jnp.float32, precision=lax.Precision.HIGHEST)
    z = z + hb[...]
    z = jnp.where(z > 0, z, ha[...] * z) * rmaskf
    buf[TOPPAD:TOPPAD + ROWS, :] = z

    z = jnp.dot(buf[TOPPAD:TOPPAD + ROWS, :], tb0[...],
                preferred_element_type=jnp.float32, precision=lax.Precision.HIGHEST) + b0b[...]
    buf2[TOPPAD:TOPPAD + ROWS, :] = z * rmaskf

    srcs = (buf2, buf3, buf2)
    dsts = (buf3, buf2, buf3)
    for li, (tw, bb) in enumerate(((t1, b1b), (t2, b2b), (t3, b3b))):
        z = jnp.zeros((ROWS, 384), jnp.float32)
        for d in range(3):
            xs = srcs[li][TOPPAD + d - 1:TOPPAD + d - 1 + ROWS, :]
            z = z + jnp.dot(xs, tw[d], preferred_element_type=jnp.float32, precision=lax.Precision.HIGHEST)
        dsts[li][TOPPAD:TOPPAD + ROWS, :] = (z + bb[...]) * rmaskf

    z = jnp.zeros((ROWS, 384), jnp.float32)
    for d in range(3):
        xs = buf3[TOPPAD + d - 1:TOPPAD + d - 1 + ROWS, :]
        z = z + jnp.dot(xs, t4[d], preferred_element_type=jnp.float32, precision=lax.Precision.HIGHEST)
    z = z + b4b[...]
    z = jnp.where(z > 0, z, b4a[...] * z) * rmaskf
    buf2[TOPPAD:TOPPAD + ROWS, :] = z

    f = flg[0, 0, 0]
    for i in range(4):
        acc = jnp.zeros((ROWS, 384), jnp.float32)
        for t, (ki, kd) in enumerate(tt_keys):
            if ki != i:
                continue
            xs = buf2[TOPPAD + kd:TOPPAD + kd + ROWS, :]
            acc = acc + jnp.dot(xs, tt[t], preferred_element_type=jnp.float32, precision=lax.Precision.HIGHEST)
        for n in range(3):
            val = (acc[0:32, n * 128:(n + 1) * 128]
                   + mb[n, :, i * 128:(i + 1) * 128]) * f
            o_ref[0, n, :, i * 128:(i + 1) * 128] = val


def _run_expert(xt, out_prev, idx, flg, wp, nf, cap):
    flg3 = flg.reshape(cap, 1, 1)
    warrs = [wp['th'], wp['hb'], wp['ha'], wp['tb0'], wp['b0b'],
             wp['t1'], wp['b1b'], wp['t2'], wp['b2b'], wp['t3'], wp['b3b'],
             wp['t4'], wp['b4b'], wp['b4a'], wp['tt'], wp['mb']]

    in_specs = [pl.BlockSpec(
        (1, 32, 96),
        lambda s, idx_ref: (jnp.minimum(idx_ref[s], NPATCH - 1), 0, 0))]
    for w in warrs:
        nd = len(w.shape)
        in_specs.append(pl.BlockSpec(
            w.shape, lambda s, idx_ref, nd=nd: tuple([0] * nd)))
    in_specs.append(pl.BlockSpec((1, 1, 1), lambda s, idx_ref: (s, 0, 0)))
    in_specs.append(pl.BlockSpec((1, 1, 8, 128),
                                 lambda s, idx_ref: (NPATCH, 0, 0, 0)))

    out_spec = pl.BlockSpec((1, 3, 32, 512),
                            lambda s, idx_ref: (idx_ref[s], 0, 0, 0))

    rows_tot = TOPPAD + ROWS + 8
    scratch = [pltpu.VMEM((rows_tot, 96), jnp.float32),
               pltpu.VMEM((rows_tot, 32 * nf), jnp.float32),
               pltpu.VMEM((rows_tot, 384), jnp.float32),
               pltpu.VMEM((rows_tot, 384), jnp.float32)]

    grid_spec = pltpu.PrefetchScalarGridSpec(
        num_scalar_prefetch=1,
        grid=(cap,),
        in_specs=in_specs,
        out_specs=out_spec,
        scratch_shapes=scratch)

    return pl.pallas_call(
        functools.partial(_expert_body, nf, wp['tt_keys']),
        grid_spec=grid_spec,
        out_shape=jax.ShapeDtypeStruct((NPATCH + 1, 3, 32, 512), jnp.float32),
        input_output_aliases={19: 0},
    )(idx, xt, *warrs, flg3, out_prev)


def kernel(x, params):
    logits = _classifier(x, params['cls'])
    idx_all, flg_all = _route(logits)

    xt = x.transpose(0, 2, 3, 1).reshape(NPATCH, 32, 96)

    out = jnp.zeros((NPATCH + 1, 3, 32, 512), jnp.float32)
    for e in range(3):
        wp = _prep_expert(params['net%d' % e], NFS[e])
        i0, i1 = BASES_E[e], BASES_E[e] + CAPS_E[e]
        out = _run_expert(xt, out, idx_all[i0:i1], flg_all[i0:i1], wp,
                          NFS[e], CAPS_E[e])
    return out.reshape(NPATCH + 1, 3, 128, 128)[:NPATCH]
